# Initial kernel scaffold; baseline (speedup 1.0000x reference)
import jax
import jax.numpy as jnp
from jax.experimental import pallas as pl


def kernel(obj_embs, obj_nums, pred_embs, pred_nums, rels, objs, params):
    raise NotImplementedError("write your pallas kernel here")


# speedup vs baseline: 1.0205x; 1.0205x over previous
import functools

import jax
import jax.numpy as jnp
from jax import lax
from jax.experimental import pallas as pl
from jax.experimental.pallas import tpu as pltpu

HID = 256
GD = 128


def _obj_fusion_body(objs_ref, obj_embs_ref, tbl_ref, wfi_ref, wfw_ref, b_ref,
                     out_ref):
    idx = objs_ref[0, 0, :]
    vp = tbl_ref.shape[0]
    onehot = (idx[:, None] == lax.broadcasted_iota(jnp.int32, (1, vp), 1)
              ).astype(jnp.float32)
    word = jnp.dot(onehot, tbl_ref[...], preferred_element_type=jnp.float32)
    h = (jnp.dot(obj_embs_ref[...], wfi_ref[...],
                 preferred_element_type=jnp.float32)
         + jnp.dot(word, wfw_ref[...], preferred_element_type=jnp.float32)
         + b_ref[...])
    out_ref[...] = jnp.maximum(h, 0.0)


def _obj_fusion(objs, obj_embs, obj_embed, wf, bf):
    o_n = obj_embs.shape[0]
    blk = 1000
    nb = o_n // blk
    vp = 256
    tbl = jnp.zeros((vp, GD), jnp.float32).at[:obj_embed.shape[0]].set(obj_embed)
    wfi = wf[:GD]
    wfw = wf[GD:]
    objs3 = objs.reshape(nb, 1, blk)
    return pl.pallas_call(
        _obj_fusion_body,
        grid=(nb,),
        in_specs=[
            pl.BlockSpec((1, 1, blk), lambda i: (i, 0, 0)),
            pl.BlockSpec((blk, GD), lambda i: (i, 0)),
            pl.BlockSpec((vp, GD), lambda i: (0, 0)),
            pl.BlockSpec((GD, GD), lambda i: (0, 0)),
            pl.BlockSpec((GD, GD), lambda i: (0, 0)),
            pl.BlockSpec((1, GD), lambda i: (0, 0)),
        ],
        out_specs=pl.BlockSpec((blk, GD), lambda i: (i, 0)),
        out_shape=jax.ShapeDtypeStruct((o_n, GD), jnp.float32),
    )(objs3, obj_embs, tbl, wfi, wfw, bf.reshape(1, GD))


def _edge_body(fuse_pred, pidx_ref, pe_ref, gs_ref, go_ref, rtbl_ref, wfi_ref,
               wfw_ref, bf_ref, w1a_ref, w1b_ref, w1c_ref, b1_ref, w2_ref,
               b2_ref, nsl_ref, nsh_ref, np_ref, nol_ref, noh_ref):
    if fuse_pred:
        pidx = pidx_ref[0, 0, :]
        vr = rtbl_ref.shape[0]
        onehot = (pidx[:, None] == lax.broadcasted_iota(jnp.int32, (1, vr), 1)
                  ).astype(jnp.float32)
        word = jnp.dot(onehot, rtbl_ref[...], preferred_element_type=jnp.float32)
        pv = (jnp.dot(pe_ref[...], wfi_ref[...],
                      preferred_element_type=jnp.float32)
              + jnp.dot(word, wfw_ref[...], preferred_element_type=jnp.float32)
              + bf_ref[...])
        pv = jnp.maximum(pv, 0.0)
    else:
        pv = pe_ref[...]
    h = (jnp.dot(gs_ref[...], w1a_ref[...], preferred_element_type=jnp.float32)
         + jnp.dot(pv, w1b_ref[...], preferred_element_type=jnp.float32)
         + jnp.dot(go_ref[...], w1c_ref[...], preferred_element_type=jnp.float32)
         + b1_ref[...])
    h = jnp.maximum(h, 0.0)
    t = jnp.maximum(
        jnp.dot(h, w2_ref[...], preferred_element_type=jnp.float32)
        + b2_ref[...], 0.0)
    nsl_ref[...] = t[:, :128]
    nsh_ref[...] = t[:, 128:256]
    np_ref[...] = t[:, 256:384]
    nol_ref[...] = t[:, 384:512]
    noh_ref[...] = t[:, 512:640]


def _edge_mlp(pidx, pe, gs, go, rel_embed, wf, bf, p, fuse_pred):
    t_n = pe.shape[0]
    blk = 640
    nb = t_n // blk
    vr = 64
    if fuse_pred:
        rtbl = jnp.zeros((vr, GD), jnp.float32).at[:rel_embed.shape[0]].set(rel_embed)
        wfi = wf[:GD]
        wfw = wf[GD:]
    else:
        rtbl = jnp.zeros((vr, GD), jnp.float32)
        wfi = jnp.zeros((GD, GD), jnp.float32)
        wfw = jnp.zeros((GD, GD), jnp.float32)
    pidx3 = pidx.reshape(nb, 1, blk)
    w1 = p['W1']
    body = functools.partial(_edge_body, fuse_pred)
    outs = pl.pallas_call(
        body,
        grid=(nb,),
        in_specs=[
            pl.BlockSpec((1, 1, blk), lambda i: (i, 0, 0)),
            pl.BlockSpec((blk, GD), lambda i: (i, 0)),
            pl.BlockSpec((blk, GD), lambda i: (i, 0)),
            pl.BlockSpec((blk, GD), lambda i: (i, 0)),
            pl.BlockSpec((vr, GD), lambda i: (0, 0)),
            pl.BlockSpec((GD, GD), lambda i: (0, 0)),
            pl.BlockSpec((GD, GD), lambda i: (0, 0)),
            pl.BlockSpec((1, GD), lambda i: (0, 0)),
            pl.BlockSpec((GD, HID), lambda i: (0, 0)),
            pl.BlockSpec((GD, HID), lambda i: (0, 0)),
            pl.BlockSpec((GD, HID), lambda i: (0, 0)),
            pl.BlockSpec((1, HID), lambda i: (0, 0)),
            pl.BlockSpec((HID, 2 * HID + GD), lambda i: (0, 0)),
            pl.BlockSpec((1, 2 * HID + GD), lambda i: (0, 0)),
        ],
        out_specs=[pl.BlockSpec((blk, GD), lambda i: (i, 0))] * 5,
        out_shape=[jax.ShapeDtypeStruct((t_n, GD), jnp.float32)] * 5,
    )(pidx3, pe, gs, go, rtbl, wfi, wfw, bf.reshape(1, GD),
      w1[:GD], w1[GD:2 * GD], w1[2 * GD:], p['b1'].reshape(1, HID),
      p['W2'], p['b2'].reshape(1, 2 * HID + GD))
    return outs


def _node_body(pl_ref, ph_ref, cnt_ref, w3l_ref, w3h_ref, b3_ref, w4_ref,
               b4_ref, out_ref):
    c = jnp.maximum(cnt_ref[...][:, :1], 1.0)
    h = (jnp.dot(pl_ref[...], w3l_ref[...], preferred_element_type=jnp.float32)
         + jnp.dot(ph_ref[...], w3h_ref[...], preferred_element_type=jnp.float32))
    h = jnp.maximum(h / c + b3_ref[...], 0.0)
    out_ref[...] = jnp.maximum(
        jnp.dot(h, w4_ref[...], preferred_element_type=jnp.float32)
        + b4_ref[...], 0.0)


def _node_mlp(pooled_lo, pooled_hi, counts, p):
    o_n = pooled_lo.shape[0]
    blk = 1000
    nb = o_n // blk
    w3 = p['W3']
    return pl.pallas_call(
        _node_body,
        grid=(nb,),
        in_specs=[
            pl.BlockSpec((blk, GD), lambda i: (i, 0)),
            pl.BlockSpec((blk, GD), lambda i: (i, 0)),
            pl.BlockSpec((blk, 16), lambda i: (i, 0)),
            pl.BlockSpec((GD, HID), lambda i: (0, 0)),
            pl.BlockSpec((GD, HID), lambda i: (0, 0)),
            pl.BlockSpec((1, HID), lambda i: (0, 0)),
            pl.BlockSpec((HID, GD), lambda i: (0, 0)),
            pl.BlockSpec((1, GD), lambda i: (0, 0)),
        ],
        out_specs=pl.BlockSpec((blk, GD), lambda i: (i, 0)),
        out_shape=jax.ShapeDtypeStruct((o_n, GD), jnp.float32),
    )(pooled_lo, pooled_hi, counts, w3[:GD], w3[GD:],
      p['b3'].reshape(1, HID), p['W4'], p['b4'].reshape(1, GD))


def _gather_rows(tbl, s_idx, o_idx):
    return tbl[s_idx], tbl[o_idx]


def _scatter_pool(o_n, s_idx, o_idx, nsl, nsh, nol, noh):
    zl = jnp.zeros((o_n, GD), jnp.float32)
    pooled_lo = zl.at[s_idx].add(nsl).at[o_idx].add(nol)
    pooled_hi = zl.at[s_idx].add(nsh).at[o_idx].add(noh)
    ones = jnp.ones((s_idx.shape[0], 16), jnp.float32)
    counts = jnp.zeros((o_n, 16), jnp.float32).at[s_idx].add(ones).at[o_idx].add(ones)
    return pooled_lo, pooled_hi, counts


def kernel(obj_embs, obj_nums, pred_embs, pred_nums, rels, objs, params):
    del obj_nums, pred_nums
    o_n = obj_embs.shape[0]
    s_idx = rels[:, 0]
    p_idx = rels[:, 1]
    o_idx = rels[:, 2]

    obj_vecs = _obj_fusion(objs, obj_embs, params['obj_embed'],
                           params['obj_fusion_W'], params['obj_fusion_b'])

    ov = obj_vecs
    pe = pred_embs
    new_p = None
    for li, fuse in [(0, True), (1, False)]:
        gp = params['gconv%d' % li]
        gs, go = _gather_rows(ov, s_idx, o_idx)
        nsl, nsh, new_p, nol, noh = _edge_mlp(
            p_idx, pe, gs, go, params['rel_embed'], params['rel_fusion_W'],
            params['rel_fusion_b'], gp, fuse)
        pooled_lo, pooled_hi, counts = _scatter_pool(
            o_n, s_idx, o_idx, nsl, nsh, nol, noh)
        ov = _node_mlp(pooled_lo, pooled_hi, counts, gp)
        pe = new_p

    return ov, new_p
